# baseline (speedup 1.0000x reference)
import functools

import jax
import jax.numpy as jnp
import numpy as np
from jax import lax
from jax.experimental import pallas as pl
from jax.experimental.pallas import tpu as pltpu
from jax.experimental.pallas import tpu_sc as plsc

N = 10000
E = 320000
DH = 128
DE = 16
G = 512

NC = 2
NS = 16
NW = NC * NS
LANES = 16

PERM_COLS = np.array([32 * (m // 32) + 16 * (m % 2) + (m % 32) // 2
                      for m in range(DH)])

CHUNK = 80
NCHUNK = E // CHUNK
ROWS_PER_SUB = 624
ROWS_LAST = N - 15 * ROWS_PER_SUB

_mesh = plsc.VectorSubcoreMesh(
    core_axis_name="c", subcore_axis_name="s", num_cores=NC, num_subcores=NS)


NBASE = NCHUNK // NW
NEXTRA = NCHUNK - NBASE * NW
NMAX = NBASE + (1 if NEXTRA else 0)
NG = (NMAX + 2 + 3) // 4


@functools.partial(
    pl.kernel,
    out_type=jax.ShapeDtypeStruct((NC, N, DH), jnp.float32),
    mesh=_mesh,
    compiler_params=pltpu.CompilerParams(needs_layout_passes=False),
    scratch_types=[
        pltpu.VMEM((4, 2, CHUNK), jnp.int32),
        pltpu.VMEM((2, CHUNK, DH), jnp.float32),
        pltpu.VMEM((2, CHUNK, DH // 2), jnp.int32),
        pltpu.VMEM_SHARED((N, DH), jnp.float32),
        pltpu.SemaphoreType.DMA,
        pltpu.SemaphoreType.DMA,
        pltpu.SemaphoreType.DMA,
        pltpu.SemaphoreType.DMA,
        pltpu.SemaphoreType.DMA,
        pltpu.SemaphoreType.DMA,
        pltpu.SemaphoreType.DMA,
        pltpu.SemaphoreType.DMA,
    ],
)
def _sc_edge_agg(eidx_hbm, e_hbm, h_hbm, out_hbm,
                 idxv, hrows, erows, aggs, sem_i0, sem_i1,
                 sem_e0, sem_e1, sem_g0, sem_g1, sem_s0, sem_s1):
    sem_i = (sem_i0, sem_i1)
    sem_e = (sem_e0, sem_e1)
    sem_g = (sem_g0, sem_g1)
    sem_s = (sem_s0, sem_s1)
    c = lax.axis_index("c")
    s = lax.axis_index("s")
    w = s * NC + c

    def _zrow(r, carry):
        for k in range(DH // LANES):
            hrows[0, r, pl.ds(k * LANES, LANES)] = jnp.zeros((LANES,), jnp.float32)
        return carry
    lax.fori_loop(0, CHUNK, _zrow, 0)
    row0 = s * ROWS_PER_SUB
    n128 = jnp.where(s == NS - 1, ROWS_LAST // CHUNK, ROWS_PER_SUB // CHUNK)

    def _zcopy(j, carry):
        pltpu.sync_copy(hrows.at[0], aggs.at[pl.ds(row0 + j * CHUNK, CHUNK)])
        return carry
    lax.fori_loop(0, n128, _zcopy, 0)

    @pl.when(s < NS - 1)
    def _ztail():
        rem = ROWS_PER_SUB - (ROWS_PER_SUB // CHUNK) * CHUNK
        pltpu.sync_copy(
            hrows.at[0, pl.ds(0, rem)],
            aggs.at[pl.ds(row0 + (ROWS_PER_SUB // CHUNK) * CHUNK, rem)])
    plsc.subcore_barrier()

    nmine = NBASE + jnp.where(w < NEXTRA, 1, 0)

    def _cid(j):
        return w + j * NW

    @pl.when(0 < nmine)
    def _pro():
        pltpu.async_copy(eidx_hbm.at[_cid(0)], idxv.at[0], sem_i[0])

    def _group(g, carry):
        for b in range(4):
            j = 4 * g + b
            p = b % 2
            np_ = 1 - p
            rn = (b + 1) % 4

            @pl.when(j + 1 < nmine)
            def _idx_next():
                pltpu.async_copy(
                    eidx_hbm.at[_cid(j + 1)], idxv.at[rn], sem_i[(b + 1) % 2])

            @pl.when(jnp.logical_and(j >= 2, j - 2 < nmine))
            def _drain():
                pltpu.make_async_copy(
                    hrows.at[p], aggs.at[idxv.at[(b - 2) % 4, 1]],
                    sem_s[p]).wait()

            @pl.when(j < nmine)
            def _launch():
                pltpu.make_async_copy(
                    eidx_hbm.at[_cid(j)], idxv.at[b], sem_i[b % 2]).wait()
                pltpu.async_copy(h_hbm.at[idxv.at[b, 0]], hrows.at[p],
                                 sem_g[p])
                pltpu.async_copy(e_hbm.at[pl.ds(_cid(j) * CHUNK, CHUNK)],
                                 erows.at[p], sem_e[p])

            @pl.when(jnp.logical_and(j >= 1, j - 1 < nmine))
            def _proc():
                cidp = _cid(j - 1)
                pltpu.make_async_copy(
                    e_hbm.at[pl.ds(cidp * CHUNK, CHUNK)], erows.at[np_],
                    sem_e[np_]).wait()
                pltpu.make_async_copy(
                    h_hbm.at[idxv.at[(b - 1) % 4, 0]], hrows.at[np_],
                    sem_g[np_]).wait()

                @plsc.parallel_loop(0, CHUNK, step=1, unroll=4)
                def _mrow(r):
                    for k in range(DH // 32):
                        eb = plsc.bitcast(erows[np_, r, pl.ds(LANES * k, LANES)],
                                          jnp.bfloat16)
                        elo, ehi = plsc.unpack(
                            eb, format=plsc.PackFormat.INTERLEAVED,
                            preferred_element_type=jnp.float32)
                        slo = pl.ds(32 * k, LANES)
                        shi = pl.ds(32 * k + LANES, LANES)
                        hrows[np_, r, slo] = jnp.maximum(
                            hrows[np_, r, slo] + elo, 0.0)
                        hrows[np_, r, shi] = jnp.maximum(
                            hrows[np_, r, shi] + ehi, 0.0)
                pltpu.async_copy(hrows.at[np_],
                                 aggs.at[idxv.at[(b - 1) % 4, 1]],
                                 sem_s[np_], add=True)
        return carry
    lax.fori_loop(0, NG, _group, 0)

    plsc.subcore_barrier()

    @pl.when(s < NS - 1)
    def _wr():
        pltpu.sync_copy(aggs.at[pl.ds(row0, ROWS_PER_SUB)],
                        out_hbm.at[c, pl.ds(row0, ROWS_PER_SUB)])

    @pl.when(s == NS - 1)
    def _wrlast():
        pltpu.sync_copy(aggs.at[pl.ds(row0, ROWS_LAST)],
                        out_hbm.at[c, pl.ds(row0, ROWS_LAST)])


PCHUNK = 80
NPCHUNK = N // PCHUNK
G_PER_SUB = G // NS


@functools.partial(
    pl.kernel,
    out_type=jax.ShapeDtypeStruct((NC, G, DH), jnp.float32),
    mesh=_mesh,
    compiler_params=pltpu.CompilerParams(needs_layout_passes=False),
    scratch_types=[
        pltpu.VMEM((PCHUNK,), jnp.int32),
        pltpu.VMEM((PCHUNK, DH), jnp.float32),
        pltpu.VMEM_SHARED((G, DH), jnp.float32),
    ],
)
def _sc_pool(batch_hbm, h_hbm, out_hbm, idxv, rows, aggs):
    c = lax.axis_index("c")
    s = lax.axis_index("s")
    w = s * NC + c

    def _zrow(r, carry):
        for k in range(DH // LANES):
            rows[r, pl.ds(k * LANES, LANES)] = jnp.zeros((LANES,), jnp.float32)
        return carry
    lax.fori_loop(0, G_PER_SUB, _zrow, 0)
    row0 = s * G_PER_SUB
    pltpu.sync_copy(rows.at[pl.ds(0, G_PER_SUB)],
                    aggs.at[pl.ds(row0, G_PER_SUB)])
    plsc.subcore_barrier()

    nbase = NPCHUNK // NW
    nmine = nbase + jnp.where(w < (NPCHUNK - nbase * NW), 1, 0)

    def _chunk(j, carry):
        base = (w + j * NW) * PCHUNK
        pltpu.sync_copy(batch_hbm.at[pl.ds(base, PCHUNK)], idxv)
        pltpu.sync_copy(h_hbm.at[pl.ds(base, PCHUNK)], rows)
        pltpu.sync_copy(rows, aggs.at[idxv], add=True)
        return carry
    lax.fori_loop(0, nmine, _chunk, 0)

    plsc.subcore_barrier()
    pltpu.sync_copy(aggs.at[pl.ds(row0, G_PER_SUB)],
                    out_hbm.at[c, pl.ds(row0, G_PER_SUB)])


def _matmul_bias(a, w, b):
    def body(a_ref, w_ref, b_ref, o_ref):
        o_ref[...] = jnp.dot(a_ref[...], w_ref[...],
                             preferred_element_type=jnp.float32) + b_ref[...]
    return pl.pallas_call(
        body,
        out_shape=jax.ShapeDtypeStruct((a.shape[0], w.shape[1]), jnp.float32),
    )(a, w, b.reshape(1, -1))


def _edge_embed(ea, w, b):
    blk = 8000

    def body(ea_ref, w_ref, b_ref, o_ref):
        o_ref[...] = (jnp.dot(ea_ref[...], w_ref[...],
                              preferred_element_type=jnp.float32)
                      + b_ref[...]).astype(jnp.bfloat16)
    return pl.pallas_call(
        body,
        grid=(E // blk,),
        in_specs=[
            pl.BlockSpec((blk, DE), lambda i: (i, 0)),
            pl.BlockSpec((DE, DH), lambda i: (0, 0)),
            pl.BlockSpec((1, DH), lambda i: (0, 0)),
        ],
        out_specs=pl.BlockSpec((blk, DH), lambda i: (i, 0)),
        out_shape=jax.ShapeDtypeStruct((E, DH), jnp.bfloat16),
    )(ea, w, b.reshape(1, -1))


def _node_update(h, aggp, w, b, gamma, beta):
    def body(h_ref, a_ref, w_ref, b_ref, g_ref, be_ref, o_ref):
        h = h_ref[...]
        hc = h + a_ref[0] + a_ref[1]
        hc = jnp.dot(hc, w_ref[...], preferred_element_type=jnp.float32) + b_ref[...]
        mu = jnp.mean(hc, axis=0, keepdims=True)
        xc = hc - mu
        var = jnp.mean(xc * xc, axis=0, keepdims=True)
        xn = xc * lax.rsqrt(var + 1e-5) * g_ref[...] + be_ref[...]
        ge = 0.5 * xn * (1.0 + lax.erf(xn * np.float32(1.0 / np.sqrt(2.0))))
        o_ref[...] = h + ge
    return pl.pallas_call(
        body,
        out_shape=jax.ShapeDtypeStruct((N, DH), jnp.float32),
    )(h, aggp, w, b.reshape(1, -1), gamma.reshape(1, -1), beta.reshape(1, -1))


def _sum2(p):
    def body(p_ref, o_ref):
        o_ref[...] = p_ref[0] + p_ref[1]
    return pl.pallas_call(
        body,
        out_shape=jax.ShapeDtypeStruct(p.shape[1:], jnp.float32),
    )(p)


def kernel(x, edge_index, edge_attr, batch, W_emb, b_emb, W_edge, b_edge,
           W_nn, b_nn, gamma, beta, W_out, b_out):
    eidx = edge_index.reshape(2, NCHUNK, CHUNK).transpose(1, 0, 2)
    L = W_edge.shape[0]
    h = _matmul_bias(x, W_emb, b_emb)
    es = [_edge_embed(edge_attr, W_edge[i][:, PERM_COLS], b_edge[i][PERM_COLS])
          for i in range(L)]

    def _pack(xb):
        return lax.bitcast_convert_type(
            xb.reshape(xb.shape[0], DH // 2, 2), jnp.int32)
    for i in range(L):
        aggp = _sc_edge_agg(eidx, _pack(es[i]), h)
        h = _node_update(h, aggp, W_nn[i], b_nn[i], gamma[i], beta[i])
    ho = _matmul_bias(h, W_out, b_out)
    poolp = _sc_pool(batch, ho)
    return _sum2(poolp)


# speedup vs baseline: 3.2775x; 3.2775x over previous
import functools

import jax
import jax.numpy as jnp
import numpy as np
from jax import lax
from jax.experimental import pallas as pl
from jax.experimental.pallas import tpu as pltpu
from jax.experimental.pallas import tpu_sc as plsc

N = 10000
E = 320000
DH = 128
DE = 16
G = 512

NC = 2
NS = 16
NW = NC * NS
LANES = 16

CHUNK = 80
NCHUNK = E // CHUNK
ROWS_PER_SUB = 624
ROWS_LAST = N - 15 * ROWS_PER_SUB

_mesh = plsc.VectorSubcoreMesh(
    core_axis_name="c", subcore_axis_name="s", num_cores=NC, num_subcores=NS)


NBASE = NCHUNK // NW
NEXTRA = NCHUNK - NBASE * NW
NMAX = NBASE + (1 if NEXTRA else 0)
NG = (NMAX + 2 + 3) // 4


@functools.partial(
    pl.kernel,
    out_type=jax.ShapeDtypeStruct((NC, N, DH), jnp.float32),
    mesh=_mesh,
    scratch_types=[
        pltpu.VMEM((4, 2, CHUNK), jnp.int32),
        pltpu.VMEM((2, CHUNK, DH), jnp.float32),
        pltpu.VMEM((2, CHUNK, DH), jnp.float32),
        pltpu.VMEM_SHARED((N, DH), jnp.float32),
        pltpu.SemaphoreType.DMA,
        pltpu.SemaphoreType.DMA,
        pltpu.SemaphoreType.DMA,
        pltpu.SemaphoreType.DMA,
        pltpu.SemaphoreType.DMA,
        pltpu.SemaphoreType.DMA,
        pltpu.SemaphoreType.DMA,
        pltpu.SemaphoreType.DMA,
    ],
)
def _sc_edge_agg(eidx_hbm, e_hbm, h_hbm, out_hbm,
                 idxv, hrows, erows, aggs, sem_i0, sem_i1,
                 sem_e0, sem_e1, sem_g0, sem_g1, sem_s0, sem_s1):
    sem_i = (sem_i0, sem_i1)
    sem_e = (sem_e0, sem_e1)
    sem_g = (sem_g0, sem_g1)
    sem_s = (sem_s0, sem_s1)
    c = lax.axis_index("c")
    s = lax.axis_index("s")
    w = s * NC + c

    def _zrow(r, carry):
        for k in range(DH // LANES):
            erows[0, r, pl.ds(k * LANES, LANES)] = jnp.zeros((LANES,), jnp.float32)
        return carry
    lax.fori_loop(0, CHUNK, _zrow, 0)
    row0 = s * ROWS_PER_SUB
    n128 = jnp.where(s == NS - 1, ROWS_LAST // CHUNK, ROWS_PER_SUB // CHUNK)

    def _zcopy(j, carry):
        pltpu.sync_copy(erows.at[0], aggs.at[pl.ds(row0 + j * CHUNK, CHUNK)])
        return carry
    lax.fori_loop(0, n128, _zcopy, 0)

    @pl.when(s < NS - 1)
    def _ztail():
        rem = ROWS_PER_SUB - (ROWS_PER_SUB // CHUNK) * CHUNK
        pltpu.sync_copy(
            erows.at[0, pl.ds(0, rem)],
            aggs.at[pl.ds(row0 + (ROWS_PER_SUB // CHUNK) * CHUNK, rem)])
    plsc.subcore_barrier()

    nmine = NBASE + jnp.where(w < NEXTRA, 1, 0)

    def _cid(j):
        return w + j * NW

    @pl.when(0 < nmine)
    def _pro():
        pltpu.async_copy(eidx_hbm.at[_cid(0)], idxv.at[0], sem_i[0])

    def _group(g, carry):
        for b in range(4):
            j = 4 * g + b
            p = b % 2
            np_ = 1 - p
            rn = (b + 1) % 4

            @pl.when(j + 1 < nmine)
            def _idx_next():
                pltpu.async_copy(
                    eidx_hbm.at[_cid(j + 1)], idxv.at[rn], sem_i[(b + 1) % 2])

            @pl.when(jnp.logical_and(j >= 2, j - 2 < nmine))
            def _drain():
                pltpu.make_async_copy(
                    erows.at[p], aggs.at[idxv.at[(b - 2) % 4, 1]],
                    sem_s[p]).wait()

            @pl.when(j < nmine)
            def _launch():
                pltpu.make_async_copy(
                    eidx_hbm.at[_cid(j)], idxv.at[b], sem_i[b % 2]).wait()
                pltpu.async_copy(h_hbm.at[idxv.at[b, 0]], hrows.at[p],
                                 sem_g[p])
                pltpu.async_copy(e_hbm.at[pl.ds(_cid(j) * CHUNK, CHUNK)],
                                 erows.at[p], sem_e[p])

            @pl.when(jnp.logical_and(j >= 1, j - 1 < nmine))
            def _proc():
                cidp = _cid(j - 1)
                pltpu.make_async_copy(
                    e_hbm.at[pl.ds(cidp * CHUNK, CHUNK)], erows.at[np_],
                    sem_e[np_]).wait()
                pltpu.make_async_copy(
                    h_hbm.at[idxv.at[(b - 1) % 4, 0]], hrows.at[np_],
                    sem_g[np_]).wait()

                @plsc.parallel_loop(0, CHUNK, step=1, unroll=8)
                def _mrow(r):
                    for k in range(DH // LANES):
                        sl = pl.ds(k * LANES, LANES)
                        erows[np_, r, sl] = jnp.maximum(
                            erows[np_, r, sl] + hrows[np_, r, sl], 0.0)
                pltpu.async_copy(erows.at[np_],
                                 aggs.at[idxv.at[(b - 1) % 4, 1]],
                                 sem_s[np_], add=True)
        return carry
    lax.fori_loop(0, NG, _group, 0)

    plsc.subcore_barrier()

    @pl.when(s < NS - 1)
    def _wr():
        pltpu.sync_copy(aggs.at[pl.ds(row0, ROWS_PER_SUB)],
                        out_hbm.at[c, pl.ds(row0, ROWS_PER_SUB)])

    @pl.when(s == NS - 1)
    def _wrlast():
        pltpu.sync_copy(aggs.at[pl.ds(row0, ROWS_LAST)],
                        out_hbm.at[c, pl.ds(row0, ROWS_LAST)])


PCHUNK = 80
NPCHUNK = N // PCHUNK
G_PER_SUB = G // NS


@functools.partial(
    pl.kernel,
    out_type=jax.ShapeDtypeStruct((NC, G, DH), jnp.float32),
    mesh=_mesh,
    scratch_types=[
        pltpu.VMEM((PCHUNK,), jnp.int32),
        pltpu.VMEM((PCHUNK, DH), jnp.float32),
        pltpu.VMEM_SHARED((G, DH), jnp.float32),
    ],
)
def _sc_pool(batch_hbm, h_hbm, out_hbm, idxv, rows, aggs):
    c = lax.axis_index("c")
    s = lax.axis_index("s")
    w = s * NC + c

    def _zrow(r, carry):
        for k in range(DH // LANES):
            rows[r, pl.ds(k * LANES, LANES)] = jnp.zeros((LANES,), jnp.float32)
        return carry
    lax.fori_loop(0, G_PER_SUB, _zrow, 0)
    row0 = s * G_PER_SUB
    pltpu.sync_copy(rows.at[pl.ds(0, G_PER_SUB)],
                    aggs.at[pl.ds(row0, G_PER_SUB)])
    plsc.subcore_barrier()

    nbase = NPCHUNK // NW
    nmine = nbase + jnp.where(w < (NPCHUNK - nbase * NW), 1, 0)

    def _chunk(j, carry):
        base = (w + j * NW) * PCHUNK
        pltpu.sync_copy(batch_hbm.at[pl.ds(base, PCHUNK)], idxv)
        pltpu.sync_copy(h_hbm.at[pl.ds(base, PCHUNK)], rows)
        pltpu.sync_copy(rows, aggs.at[idxv], add=True)
        return carry
    lax.fori_loop(0, nmine, _chunk, 0)

    plsc.subcore_barrier()
    pltpu.sync_copy(aggs.at[pl.ds(row0, G_PER_SUB)],
                    out_hbm.at[c, pl.ds(row0, G_PER_SUB)])


def _matmul_bias(a, w, b):
    def body(a_ref, w_ref, b_ref, o_ref):
        o_ref[...] = jnp.dot(a_ref[...], w_ref[...],
                             preferred_element_type=jnp.float32) + b_ref[...]
    return pl.pallas_call(
        body,
        out_shape=jax.ShapeDtypeStruct((a.shape[0], w.shape[1]), jnp.float32),
    )(a, w, b.reshape(1, -1))


def _edge_embed(ea, w, b):
    blk = 8000

    def body(ea_ref, w_ref, b_ref, o_ref):
        o_ref[...] = jnp.dot(ea_ref[...], w_ref[...],
                             preferred_element_type=jnp.float32) + b_ref[...]
    return pl.pallas_call(
        body,
        grid=(E // blk,),
        in_specs=[
            pl.BlockSpec((blk, DE), lambda i: (i, 0)),
            pl.BlockSpec((DE, DH), lambda i: (0, 0)),
            pl.BlockSpec((1, DH), lambda i: (0, 0)),
        ],
        out_specs=pl.BlockSpec((blk, DH), lambda i: (i, 0)),
        out_shape=jax.ShapeDtypeStruct((E, DH), jnp.float32),
    )(ea, w, b.reshape(1, -1))


def _node_update(h, aggp, w, b, gamma, beta):
    def body(h_ref, a_ref, w_ref, b_ref, g_ref, be_ref, o_ref):
        h = h_ref[...]
        hc = h + a_ref[0] + a_ref[1]
        hc = jnp.dot(hc, w_ref[...], preferred_element_type=jnp.float32) + b_ref[...]
        mu = jnp.mean(hc, axis=0, keepdims=True)
        xc = hc - mu
        var = jnp.mean(xc * xc, axis=0, keepdims=True)
        xn = xc * lax.rsqrt(var + 1e-5) * g_ref[...] + be_ref[...]
        ge = 0.5 * xn * (1.0 + lax.erf(xn * np.float32(1.0 / np.sqrt(2.0))))
        o_ref[...] = h + ge
    return pl.pallas_call(
        body,
        out_shape=jax.ShapeDtypeStruct((N, DH), jnp.float32),
    )(h, aggp, w, b.reshape(1, -1), gamma.reshape(1, -1), beta.reshape(1, -1))


def _sum2(p):
    def body(p_ref, o_ref):
        o_ref[...] = p_ref[0] + p_ref[1]
    return pl.pallas_call(
        body,
        out_shape=jax.ShapeDtypeStruct(p.shape[1:], jnp.float32),
    )(p)


def kernel(x, edge_index, edge_attr, batch, W_emb, b_emb, W_edge, b_edge,
           W_nn, b_nn, gamma, beta, W_out, b_out):
    eidx = edge_index.reshape(2, NCHUNK, CHUNK).transpose(1, 0, 2)
    h = _matmul_bias(x, W_emb, b_emb)
    es = [_edge_embed(edge_attr, W_edge[i], b_edge[i])
          for i in range(W_edge.shape[0])]
    for i in range(W_edge.shape[0]):
        aggp = _sc_edge_agg(eidx, es[i], h)
        h = _node_update(h, aggp, W_nn[i], b_nn[i], gamma[i], beta[i])
    ho = _matmul_bias(h, W_out, b_out)
    poolp = _sc_pool(batch, ho)
    return _sum2(poolp)


# speedup vs baseline: 3.3773x; 1.0304x over previous
import functools

import jax
import jax.numpy as jnp
import numpy as np
from jax import lax
from jax.experimental import pallas as pl
from jax.experimental.pallas import tpu as pltpu
from jax.experimental.pallas import tpu_sc as plsc

N = 10000
E = 320000
DH = 128
DE = 16
G = 512

NC = 2
NS = 16
NW = NC * NS
LANES = 16

CHUNK = 80
NCHUNK = E // CHUNK
ROWS_PER_SUB = 624
ROWS_LAST = N - 15 * ROWS_PER_SUB

_mesh = plsc.VectorSubcoreMesh(
    core_axis_name="c", subcore_axis_name="s", num_cores=NC, num_subcores=NS)


NBASE = NCHUNK // NW
NEXTRA = NCHUNK - NBASE * NW
NMAX = NBASE + (1 if NEXTRA else 0)
NG = (NMAX + 2 + 3) // 4


@functools.partial(
    pl.kernel,
    out_type=jax.ShapeDtypeStruct((NC, N, DH), jnp.float32),
    mesh=_mesh,
    scratch_types=[
        pltpu.VMEM((4, 2, CHUNK), jnp.int32),
        pltpu.VMEM((2, CHUNK, DH), jnp.float32),
        pltpu.VMEM((2, CHUNK, DH), jnp.float32),
        pltpu.VMEM_SHARED((N, DH), jnp.float32),
        pltpu.SemaphoreType.DMA,
        pltpu.SemaphoreType.DMA,
        pltpu.SemaphoreType.DMA,
        pltpu.SemaphoreType.DMA,
        pltpu.SemaphoreType.DMA,
        pltpu.SemaphoreType.DMA,
        pltpu.SemaphoreType.DMA,
        pltpu.SemaphoreType.DMA,
    ],
)
def _sc_edge_agg(eidx_hbm, e_hbm, h_hbm, out_hbm,
                 idxv, hrows, erows, aggs, sem_i0, sem_i1,
                 sem_e0, sem_e1, sem_g0, sem_g1, sem_s0, sem_s1):
    sem_i = (sem_i0, sem_i1)
    sem_e = (sem_e0, sem_e1)
    sem_g = (sem_g0, sem_g1)
    sem_s = (sem_s0, sem_s1)
    c = lax.axis_index("c")
    s = lax.axis_index("s")
    w = s * NC + c

    def _zrow(r, carry):
        for k in range(DH // LANES):
            erows[0, r, pl.ds(k * LANES, LANES)] = jnp.zeros((LANES,), jnp.float32)
        return carry
    lax.fori_loop(0, CHUNK, _zrow, 0)
    row0 = s * ROWS_PER_SUB
    n128 = jnp.where(s == NS - 1, ROWS_LAST // CHUNK, ROWS_PER_SUB // CHUNK)

    def _zcopy(j, carry):
        pltpu.sync_copy(erows.at[0], aggs.at[pl.ds(row0 + j * CHUNK, CHUNK)])
        return carry
    lax.fori_loop(0, n128, _zcopy, 0)

    @pl.when(s < NS - 1)
    def _ztail():
        rem = ROWS_PER_SUB - (ROWS_PER_SUB // CHUNK) * CHUNK
        pltpu.sync_copy(
            erows.at[0, pl.ds(0, rem)],
            aggs.at[pl.ds(row0 + (ROWS_PER_SUB // CHUNK) * CHUNK, rem)])
    plsc.subcore_barrier()

    nmine = NBASE + jnp.where(w < NEXTRA, 1, 0)

    def _cid(j):
        return w + j * NW

    @pl.when(0 < nmine)
    def _pro():
        pltpu.async_copy(eidx_hbm.at[_cid(0)], idxv.at[0], sem_i[0])

    def _group(g, carry):
        for b in range(4):
            j = 4 * g + b
            p = b % 2
            np_ = 1 - p
            rn = (b + 1) % 4

            @pl.when(j + 1 < nmine)
            def _idx_next():
                pltpu.async_copy(
                    eidx_hbm.at[_cid(j + 1)], idxv.at[rn], sem_i[(b + 1) % 2])

            @pl.when(jnp.logical_and(j >= 2, j - 2 < nmine))
            def _drain():
                pltpu.make_async_copy(
                    erows.at[p], aggs.at[idxv.at[(b - 2) % 4, 1]],
                    sem_s[p]).wait()

            @pl.when(j < nmine)
            def _launch():
                pltpu.make_async_copy(
                    eidx_hbm.at[_cid(j)], idxv.at[b], sem_i[b % 2]).wait()
                pltpu.async_copy(h_hbm.at[idxv.at[b, 0]], hrows.at[p],
                                 sem_g[p])
                pltpu.async_copy(e_hbm.at[pl.ds(_cid(j) * CHUNK, CHUNK)],
                                 erows.at[p], sem_e[p])

            @pl.when(jnp.logical_and(j >= 1, j - 1 < nmine))
            def _proc():
                cidp = _cid(j - 1)
                pltpu.make_async_copy(
                    e_hbm.at[pl.ds(cidp * CHUNK, CHUNK)], erows.at[np_],
                    sem_e[np_]).wait()
                pltpu.make_async_copy(
                    h_hbm.at[idxv.at[(b - 1) % 4, 0]], hrows.at[np_],
                    sem_g[np_]).wait()

                @plsc.parallel_loop(0, CHUNK, step=1, unroll=4)
                def _mrow(r):
                    for k in range(DH // LANES):
                        sl = pl.ds(k * LANES, LANES)
                        erows[np_, r, sl] = jnp.maximum(
                            erows[np_, r, sl] + hrows[np_, r, sl], 0.0)
                pltpu.async_copy(erows.at[np_],
                                 aggs.at[idxv.at[(b - 1) % 4, 1]],
                                 sem_s[np_], add=True)
        return carry
    lax.fori_loop(0, NG, _group, 0)

    plsc.subcore_barrier()

    @pl.when(s < NS - 1)
    def _wr():
        pltpu.sync_copy(aggs.at[pl.ds(row0, ROWS_PER_SUB)],
                        out_hbm.at[c, pl.ds(row0, ROWS_PER_SUB)])

    @pl.when(s == NS - 1)
    def _wrlast():
        pltpu.sync_copy(aggs.at[pl.ds(row0, ROWS_LAST)],
                        out_hbm.at[c, pl.ds(row0, ROWS_LAST)])


PCHUNK = 80
NPCHUNK = N // PCHUNK
G_PER_SUB = G // NS


@functools.partial(
    pl.kernel,
    out_type=jax.ShapeDtypeStruct((NC, G, DH), jnp.float32),
    mesh=_mesh,
    scratch_types=[
        pltpu.VMEM((PCHUNK,), jnp.int32),
        pltpu.VMEM((PCHUNK, DH), jnp.float32),
        pltpu.VMEM_SHARED((G, DH), jnp.float32),
    ],
)
def _sc_pool(batch_hbm, h_hbm, out_hbm, idxv, rows, aggs):
    c = lax.axis_index("c")
    s = lax.axis_index("s")
    w = s * NC + c

    def _zrow(r, carry):
        for k in range(DH // LANES):
            rows[r, pl.ds(k * LANES, LANES)] = jnp.zeros((LANES,), jnp.float32)
        return carry
    lax.fori_loop(0, G_PER_SUB, _zrow, 0)
    row0 = s * G_PER_SUB
    pltpu.sync_copy(rows.at[pl.ds(0, G_PER_SUB)],
                    aggs.at[pl.ds(row0, G_PER_SUB)])
    plsc.subcore_barrier()

    nbase = NPCHUNK // NW
    nmine = nbase + jnp.where(w < (NPCHUNK - nbase * NW), 1, 0)

    def _chunk(j, carry):
        base = (w + j * NW) * PCHUNK
        pltpu.sync_copy(batch_hbm.at[pl.ds(base, PCHUNK)], idxv)
        pltpu.sync_copy(h_hbm.at[pl.ds(base, PCHUNK)], rows)
        pltpu.sync_copy(rows, aggs.at[idxv], add=True)
        return carry
    lax.fori_loop(0, nmine, _chunk, 0)

    plsc.subcore_barrier()
    pltpu.sync_copy(aggs.at[pl.ds(row0, G_PER_SUB)],
                    out_hbm.at[c, pl.ds(row0, G_PER_SUB)])


def _matmul_bias(a, w, b):
    def body(a_ref, w_ref, b_ref, o_ref):
        o_ref[...] = jnp.dot(a_ref[...], w_ref[...],
                             preferred_element_type=jnp.float32) + b_ref[...]
    return pl.pallas_call(
        body,
        out_shape=jax.ShapeDtypeStruct((a.shape[0], w.shape[1]), jnp.float32),
    )(a, w, b.reshape(1, -1))


def _edge_embed(ea, w, b):
    blk = 8000

    def body(ea_ref, w_ref, b_ref, o_ref):
        o_ref[...] = jnp.dot(ea_ref[...], w_ref[...],
                             preferred_element_type=jnp.float32) + b_ref[...]
    return pl.pallas_call(
        body,
        grid=(E // blk,),
        in_specs=[
            pl.BlockSpec((blk, DE), lambda i: (i, 0)),
            pl.BlockSpec((DE, DH), lambda i: (0, 0)),
            pl.BlockSpec((1, DH), lambda i: (0, 0)),
        ],
        out_specs=pl.BlockSpec((blk, DH), lambda i: (i, 0)),
        out_shape=jax.ShapeDtypeStruct((E, DH), jnp.float32),
    )(ea, w, b.reshape(1, -1))


def _node_update(h, aggp, w, b, gamma, beta):
    def body(h_ref, a_ref, w_ref, b_ref, g_ref, be_ref, o_ref):
        h = h_ref[...]
        hc = h + a_ref[0] + a_ref[1]
        hc = jnp.dot(hc, w_ref[...], preferred_element_type=jnp.float32) + b_ref[...]
        mu = jnp.mean(hc, axis=0, keepdims=True)
        xc = hc - mu
        var = jnp.mean(xc * xc, axis=0, keepdims=True)
        xn = xc * lax.rsqrt(var + 1e-5) * g_ref[...] + be_ref[...]
        ge = 0.5 * xn * (1.0 + lax.erf(xn * np.float32(1.0 / np.sqrt(2.0))))
        o_ref[...] = h + ge
    return pl.pallas_call(
        body,
        out_shape=jax.ShapeDtypeStruct((N, DH), jnp.float32),
    )(h, aggp, w, b.reshape(1, -1), gamma.reshape(1, -1), beta.reshape(1, -1))


def _sum2(p):
    def body(p_ref, o_ref):
        o_ref[...] = p_ref[0] + p_ref[1]
    return pl.pallas_call(
        body,
        out_shape=jax.ShapeDtypeStruct(p.shape[1:], jnp.float32),
    )(p)


def kernel(x, edge_index, edge_attr, batch, W_emb, b_emb, W_edge, b_edge,
           W_nn, b_nn, gamma, beta, W_out, b_out):
    eidx = edge_index.reshape(2, NCHUNK, CHUNK).transpose(1, 0, 2)
    h = _matmul_bias(x, W_emb, b_emb)
    es = [_edge_embed(edge_attr, W_edge[i], b_edge[i])
          for i in range(W_edge.shape[0])]
    for i in range(W_edge.shape[0]):
        aggp = _sc_edge_agg(eidx, es[i], h)
        h = _node_update(h, aggp, W_nn[i], b_nn[i], gamma[i], beta[i])
    ho = _matmul_bias(h, W_out, b_out)
    poolp = _sc_pool(batch, ho)
    return _sum2(poolp)


# speedup vs baseline: 3.3824x; 1.0015x over previous
import functools

import jax
import jax.numpy as jnp
import numpy as np
from jax import lax
from jax.experimental import pallas as pl
from jax.experimental.pallas import tpu as pltpu
from jax.experimental.pallas import tpu_sc as plsc

N = 10000
E = 320000
DH = 128
DE = 16
G = 512

NC = 2
NS = 16
NW = NC * NS
LANES = 16

CHUNK = 80
NCHUNK = E // CHUNK
ROWS_PER_SUB = 624
ROWS_LAST = N - 15 * ROWS_PER_SUB

_mesh = plsc.VectorSubcoreMesh(
    core_axis_name="c", subcore_axis_name="s", num_cores=NC, num_subcores=NS)


NBASE = NCHUNK // NW
NEXTRA = NCHUNK - NBASE * NW
NMAX = NBASE + (1 if NEXTRA else 0)
NG = (NMAX + 2 + 3) // 4


@functools.partial(
    pl.kernel,
    out_type=jax.ShapeDtypeStruct((NC, N, DH), jnp.float32),
    mesh=_mesh,
    scratch_types=[
        pltpu.VMEM((4, 2, CHUNK), jnp.int32),
        pltpu.VMEM((2, CHUNK, DH), jnp.float32),
        pltpu.VMEM((2, CHUNK, DH), jnp.float32),
        pltpu.VMEM_SHARED((N, DH), jnp.float32),
        pltpu.SemaphoreType.DMA,
        pltpu.SemaphoreType.DMA,
        pltpu.SemaphoreType.DMA,
        pltpu.SemaphoreType.DMA,
        pltpu.SemaphoreType.DMA,
        pltpu.SemaphoreType.DMA,
        pltpu.SemaphoreType.DMA,
        pltpu.SemaphoreType.DMA,
    ],
)
def _sc_edge_agg(eidx_hbm, e_hbm, h_hbm, out_hbm,
                 idxv, hrows, erows, aggs, sem_i0, sem_i1,
                 sem_e0, sem_e1, sem_g0, sem_g1, sem_s0, sem_s1):
    sem_i = (sem_i0, sem_i1)
    sem_e = (sem_e0, sem_e1)
    sem_g = (sem_g0, sem_g1)
    sem_s = (sem_s0, sem_s1)
    c = lax.axis_index("c")
    s = lax.axis_index("s")
    w = s * NC + c

    def _zrow(r, carry):
        for k in range(DH // LANES):
            erows[0, r, pl.ds(k * LANES, LANES)] = jnp.zeros((LANES,), jnp.float32)
        return carry
    lax.fori_loop(0, CHUNK, _zrow, 0)
    row0 = s * ROWS_PER_SUB
    n128 = jnp.where(s == NS - 1, ROWS_LAST // CHUNK, ROWS_PER_SUB // CHUNK)

    def _zcopy(j, carry):
        pltpu.sync_copy(erows.at[0], aggs.at[pl.ds(row0 + j * CHUNK, CHUNK)])
        return carry
    lax.fori_loop(0, n128, _zcopy, 0)

    @pl.when(s < NS - 1)
    def _ztail():
        rem = ROWS_PER_SUB - (ROWS_PER_SUB // CHUNK) * CHUNK
        pltpu.sync_copy(
            erows.at[0, pl.ds(0, rem)],
            aggs.at[pl.ds(row0 + (ROWS_PER_SUB // CHUNK) * CHUNK, rem)])
    plsc.subcore_barrier()

    nmine = NBASE + jnp.where(w < NEXTRA, 1, 0)

    def _cid(j):
        return w + j * NW

    @pl.when(0 < nmine)
    def _pro():
        pltpu.async_copy(eidx_hbm.at[_cid(0)], idxv.at[0], sem_i[0])

    def _group(g, carry):
        for b in range(4):
            j = 4 * g + b
            p = b % 2
            np_ = 1 - p
            rn = (b + 1) % 4

            @pl.when(j + 1 < nmine)
            def _idx_next():
                pltpu.async_copy(
                    eidx_hbm.at[_cid(j + 1)], idxv.at[rn], sem_i[(b + 1) % 2])

            @pl.when(jnp.logical_and(j >= 2, j - 2 < nmine))
            def _drain():
                pltpu.make_async_copy(
                    erows.at[p], aggs.at[idxv.at[(b - 2) % 4, 1]],
                    sem_s[p]).wait()

            @pl.when(j < nmine)
            def _launch():
                pltpu.async_copy(e_hbm.at[pl.ds(_cid(j) * CHUNK, CHUNK)],
                                 erows.at[p], sem_e[p])
                pltpu.make_async_copy(
                    eidx_hbm.at[_cid(j)], idxv.at[b], sem_i[b % 2]).wait()
                pltpu.async_copy(h_hbm.at[idxv.at[b, 0]], hrows.at[p],
                                 sem_g[p])

            @pl.when(jnp.logical_and(j >= 1, j - 1 < nmine))
            def _proc():
                cidp = _cid(j - 1)
                pltpu.make_async_copy(
                    e_hbm.at[pl.ds(cidp * CHUNK, CHUNK)], erows.at[np_],
                    sem_e[np_]).wait()
                pltpu.make_async_copy(
                    h_hbm.at[idxv.at[(b - 1) % 4, 0]], hrows.at[np_],
                    sem_g[np_]).wait()

                @plsc.parallel_loop(0, CHUNK, step=1, unroll=4)
                def _mrow(r):
                    for k in range(DH // LANES):
                        sl = pl.ds(k * LANES, LANES)
                        erows[np_, r, sl] = jnp.maximum(
                            erows[np_, r, sl] + hrows[np_, r, sl], 0.0)
                pltpu.async_copy(erows.at[np_],
                                 aggs.at[idxv.at[(b - 1) % 4, 1]],
                                 sem_s[np_], add=True)
        return carry
    lax.fori_loop(0, NG, _group, 0)

    plsc.subcore_barrier()

    @pl.when(s < NS - 1)
    def _wr():
        pltpu.sync_copy(aggs.at[pl.ds(row0, ROWS_PER_SUB)],
                        out_hbm.at[c, pl.ds(row0, ROWS_PER_SUB)])

    @pl.when(s == NS - 1)
    def _wrlast():
        pltpu.sync_copy(aggs.at[pl.ds(row0, ROWS_LAST)],
                        out_hbm.at[c, pl.ds(row0, ROWS_LAST)])


PCHUNK = 80
NPCHUNK = N // PCHUNK
G_PER_SUB = G // NS


@functools.partial(
    pl.kernel,
    out_type=jax.ShapeDtypeStruct((NC, G, DH), jnp.float32),
    mesh=_mesh,
    scratch_types=[
        pltpu.VMEM((PCHUNK,), jnp.int32),
        pltpu.VMEM((PCHUNK, DH), jnp.float32),
        pltpu.VMEM_SHARED((G, DH), jnp.float32),
    ],
)
def _sc_pool(batch_hbm, h_hbm, out_hbm, idxv, rows, aggs):
    c = lax.axis_index("c")
    s = lax.axis_index("s")
    w = s * NC + c

    def _zrow(r, carry):
        for k in range(DH // LANES):
            rows[r, pl.ds(k * LANES, LANES)] = jnp.zeros((LANES,), jnp.float32)
        return carry
    lax.fori_loop(0, G_PER_SUB, _zrow, 0)
    row0 = s * G_PER_SUB
    pltpu.sync_copy(rows.at[pl.ds(0, G_PER_SUB)],
                    aggs.at[pl.ds(row0, G_PER_SUB)])
    plsc.subcore_barrier()

    nbase = NPCHUNK // NW
    nmine = nbase + jnp.where(w < (NPCHUNK - nbase * NW), 1, 0)

    def _chunk(j, carry):
        base = (w + j * NW) * PCHUNK
        pltpu.sync_copy(batch_hbm.at[pl.ds(base, PCHUNK)], idxv)
        pltpu.sync_copy(h_hbm.at[pl.ds(base, PCHUNK)], rows)
        pltpu.sync_copy(rows, aggs.at[idxv], add=True)
        return carry
    lax.fori_loop(0, nmine, _chunk, 0)

    plsc.subcore_barrier()
    pltpu.sync_copy(aggs.at[pl.ds(row0, G_PER_SUB)],
                    out_hbm.at[c, pl.ds(row0, G_PER_SUB)])


def _matmul_bias(a, w, b):
    def body(a_ref, w_ref, b_ref, o_ref):
        o_ref[...] = jnp.dot(a_ref[...], w_ref[...],
                             preferred_element_type=jnp.float32) + b_ref[...]
    return pl.pallas_call(
        body,
        out_shape=jax.ShapeDtypeStruct((a.shape[0], w.shape[1]), jnp.float32),
    )(a, w, b.reshape(1, -1))


def _edge_embed(ea, w, b):
    blk = 8000

    def body(ea_ref, w_ref, b_ref, o_ref):
        o_ref[...] = jnp.dot(ea_ref[...], w_ref[...],
                             preferred_element_type=jnp.float32) + b_ref[...]
    return pl.pallas_call(
        body,
        grid=(E // blk,),
        in_specs=[
            pl.BlockSpec((blk, DE), lambda i: (i, 0)),
            pl.BlockSpec((DE, DH), lambda i: (0, 0)),
            pl.BlockSpec((1, DH), lambda i: (0, 0)),
        ],
        out_specs=pl.BlockSpec((blk, DH), lambda i: (i, 0)),
        out_shape=jax.ShapeDtypeStruct((E, DH), jnp.float32),
    )(ea, w, b.reshape(1, -1))


def _node_update(h, aggp, w, b, gamma, beta):
    def body(h_ref, a_ref, w_ref, b_ref, g_ref, be_ref, o_ref):
        h = h_ref[...]
        hc = h + a_ref[0] + a_ref[1]
        hc = jnp.dot(hc, w_ref[...], preferred_element_type=jnp.float32) + b_ref[...]
        mu = jnp.mean(hc, axis=0, keepdims=True)
        xc = hc - mu
        var = jnp.mean(xc * xc, axis=0, keepdims=True)
        xn = xc * lax.rsqrt(var + 1e-5) * g_ref[...] + be_ref[...]
        ge = 0.5 * xn * (1.0 + lax.erf(xn * np.float32(1.0 / np.sqrt(2.0))))
        o_ref[...] = h + ge
    return pl.pallas_call(
        body,
        out_shape=jax.ShapeDtypeStruct((N, DH), jnp.float32),
    )(h, aggp, w, b.reshape(1, -1), gamma.reshape(1, -1), beta.reshape(1, -1))


def _sum2(p):
    def body(p_ref, o_ref):
        o_ref[...] = p_ref[0] + p_ref[1]
    return pl.pallas_call(
        body,
        out_shape=jax.ShapeDtypeStruct(p.shape[1:], jnp.float32),
    )(p)


def kernel(x, edge_index, edge_attr, batch, W_emb, b_emb, W_edge, b_edge,
           W_nn, b_nn, gamma, beta, W_out, b_out):
    eidx = edge_index.reshape(2, NCHUNK, CHUNK).transpose(1, 0, 2)
    h = _matmul_bias(x, W_emb, b_emb)
    es = [_edge_embed(edge_attr, W_edge[i], b_edge[i])
          for i in range(W_edge.shape[0])]
    for i in range(W_edge.shape[0]):
        aggp = _sc_edge_agg(eidx, es[i], h)
        h = _node_update(h, aggp, W_nn[i], b_nn[i], gamma[i], beta[i])
    ho = _matmul_bias(h, W_out, b_out)
    poolp = _sc_pool(batch, ho)
    return _sum2(poolp)
